# baseline (speedup 1.0000x reference)
import functools

import jax
import jax.numpy as jnp
from jax import lax
from jax.experimental import pallas as pl
from jax.experimental.pallas import tpu as pltpu
from jax.experimental.pallas import tpu_sc as plsc

_NC = 2
_NS = 16
_NW = _NC * _NS
_L = 16
_BT = 128


@functools.partial(jax.jit, static_argnames=("b", "s", "d"))
def _emb_lookup(xt, feature_matrix, b, s, d):
    n_bt = b // _BT
    ft = d // 8
    st_n = s // 8
    assert n_bt == _NW and d % 8 == 0
    v = feature_matrix.shape[0]
    mesh = plsc.VectorSubcoreMesh(core_axis_name="c", subcore_axis_name="s")

    @functools.partial(
        pl.kernel,
        mesh=mesh,
        compiler_params=pltpu.CompilerParams(
            use_tc_tiling_on_sc=False, needs_layout_passes=False
        ),
        out_type=jax.ShapeDtypeStruct((s, ft, n_bt, 8, _BT), jnp.float32),
        scratch_types=[
            pltpu.VMEM((v, d), jnp.float32),
            pltpu.VMEM((st_n, 8, _BT), jnp.int32),
            pltpu.VMEM((4, ft, 8, _BT), jnp.float32),
            pltpu.SemaphoreType.DMA,
            pltpu.SemaphoreType.DMA,
            pltpu.SemaphoreType.DMA,
            pltpu.SemaphoreType.DMA,
        ],
    )
    def emb(x_hbm, tab_hbm, out_hbm, tab_v, idx_v, tile_v,
            osem0, osem1, osem2, osem3):
        wid = lax.axis_index("s") * _NC + lax.axis_index("c")
        pltpu.sync_copy(tab_hbm, tab_v)
        pltpu.sync_copy(x_hbm.at[:, wid], idx_v)

        f_vecs = [jnp.full((_L,), f, jnp.int32) for f in range(d)]
        osems = (osem0, osem1, osem2, osem3)
        depth = 4

        def unit(u, slot):
            @pl.when(u >= depth)
            def _():
                pltpu.make_async_copy(
                    tile_v.at[slot],
                    out_hbm.at[0, pl.ds(0, ft), wid],
                    osems[slot],
                ).wait()

            st = u // 8
            si = lax.rem(u, 8)
            idx_gs = [
                idx_v[st, si, pl.ds(g * _L, _L)] for g in range(_BT // _L)
            ]
            pairs = [(f, g) for g in range(_BT // _L) for f in range(d)]
            lag = 8
            pend = {}
            for i, (f, g) in enumerate(pairs):
                pend[i] = plsc.load_gather(tab_v, [idx_gs[g], f_vecs[f]])
                if i >= lag:
                    pf, pg = pairs[i - lag]
                    tile_v[slot, pf // 8, pf % 8, pl.ds(pg * _L, _L)] = (
                        pend.pop(i - lag)
                    )
            for i in range(len(pairs) - lag, len(pairs)):
                pf, pg = pairs[i]
                tile_v[slot, pf // 8, pf % 8, pl.ds(pg * _L, _L)] = (
                    pend.pop(i)
                )

            pltpu.async_copy(
                tile_v.at[slot],
                out_hbm.at[u, pl.ds(0, ft), wid],
                osems[slot],
            )

        def body(t, carry):
            for k in range(depth):
                unit(depth * t + k, k)
            return carry

        lax.fori_loop(0, s // depth, body, 0)
        for slot in range(depth):
            pltpu.make_async_copy(
                tile_v.at[slot],
                out_hbm.at[0, pl.ds(0, ft), wid],
                osems[slot],
            ).wait()

    return emb(xt, feature_matrix)


def kernel(x, feature_matrix):
    b, s = x.shape
    v, d = feature_matrix.shape
    assert b % _BT == 0 and b // _BT == _NW and s % 8 == 0
    xt = (
        x.astype(jnp.int32)
        .T.reshape(s // 8, 8, _NW, _BT)
        .transpose(0, 2, 1, 3)
    )
    out5 = _emb_lookup(xt, feature_matrix, b, s, d)
    return out5.transpose(2, 4, 0, 1, 3).reshape(b, s, d)
